# baseline (speedup 1.0000x reference)
import functools

import jax
import jax.numpy as jnp
from jax.experimental import pallas as pl
from jax.experimental.pallas import tpu as pltpu


def _gelu(x):
    c = 0.7978845608028654
    return 0.5 * x * (1.0 + jnp.tanh(c * (x + 0.044715 * x * x * x)))


_TRANS_RHS = (((1,), (1,)), ((), ()))


def _gcn_fused_kernel(adj_ref, x_ref, wdr_ref, w1_ref, g_ref, b_ref, w2_ref,
                      out_ref, *, G, N):
    H = wdr_ref.shape[0]
    GN = G * N
    mm = wdr_ref.dtype
    wdr = wdr_ref[...]
    x0 = x_ref[...].reshape(GN, H)

    adj2 = adj_ref[...].reshape(GN, N)
    sel = jax.lax.broadcasted_iota(jnp.int32, (N, GN), 1)
    lane = jax.lax.broadcasted_iota(jnp.int32, (N, GN), 0)
    expand = (sel % N == lane).astype(jnp.float32)
    tiled = jnp.dot(adj2, expand,
                    preferred_element_type=jnp.float32)
    row = jax.lax.broadcasted_iota(jnp.int32, (GN, GN), 0)
    col = jax.lax.broadcasted_iota(jnp.int32, (GN, GN), 1)
    bd = jnp.where((row // N) == (col // N), tiled, 0.0).astype(mm)
    _T_LHS = (((0,), (0,)), ((), ()))

    def layer(x_bf):
        y = jnp.dot(x_bf, wdr,
                    preferred_element_type=jnp.float32)
        d = _gelu(y[:, :H].astype(mm))
        r = y[:, H:]
        diff = jax.lax.dot_general(bd, d, _T_LHS,
                                   preferred_element_type=jnp.float32)
        return _gelu((r + diff).astype(mm))

    x = layer(layer(x0.astype(mm)))

    h = _gelu(jnp.dot(x, w1_ref[...],
                      preferred_element_type=jnp.float32).astype(mm))
    h = h.astype(jnp.float32)
    mu = jnp.mean(h, axis=-1, keepdims=True)
    var = jnp.mean((h - mu) ** 2, axis=-1, keepdims=True)
    h = ((h - mu) * jax.lax.rsqrt(var + 1e-5)
         * g_ref[...].astype(jnp.float32) + b_ref[...].astype(jnp.float32))
    out_ref[0] = jax.lax.dot_general(w2_ref[...], h.astype(mm), _TRANS_RHS,
                                     preferred_element_type=jnp.float32)


def _choose_group(B, N, target_rows=128):
    per = max(1, target_rows // N)
    if B >= 2:
        per = min(per, B // 2)
    per = max(1, per)
    while B % per:
        per -= 1
    return per


def _gcn_pallas(adj_b, sem_b, wdr_bf, w1_bf, ln_g, ln_b, w2_bf, *, G, N):
    B = adj_b.shape[0]
    H = sem_b.shape[-1]
    Bg, GN = B // G, G * N
    const2 = lambda b: (0, 0)
    once = pl.Buffered(buffer_count=1)
    return pl.pallas_call(
        functools.partial(_gcn_fused_kernel, G=G, N=N),
        out_shape=jax.ShapeDtypeStruct((Bg, 1, GN), jnp.float32),
        grid_spec=pltpu.PrefetchScalarGridSpec(
            num_scalar_prefetch=0,
            grid=(Bg,),
            in_specs=[
                pl.BlockSpec((G, N, N), lambda b: (b, 0, 0)),
                pl.BlockSpec((G, N, H), lambda b: (b, 0, 0)),
                pl.BlockSpec((H, 2 * H), const2, pipeline_mode=once),
                pl.BlockSpec((H, H), const2, pipeline_mode=once),
                pl.BlockSpec((1, H), const2, pipeline_mode=once),
                pl.BlockSpec((1, H), const2, pipeline_mode=once),
                pl.BlockSpec((1, H), const2, pipeline_mode=once),
            ],
            out_specs=pl.BlockSpec((1, 1, GN), lambda b: (b, 0, 0)),
        ),
        compiler_params=pltpu.CompilerParams(
            dimension_semantics=("parallel",),
            vmem_limit_bytes=64 * 1024 * 1024,
        ),
    )(adj_b, sem_b, wdr_bf, w1_bf, ln_g, ln_b, w2_bf)


def kernel(adj_b, sem_b, wdr_t, w1_t, ln_g, ln_b, w2):
    B, N, _ = adj_b.shape
    H = sem_b.shape[-1]
    G = _choose_group(B, N)
    mm = jnp.bfloat16
    out = _gcn_pallas(adj_b, sem_b, wdr_t.astype(mm), w1_t.astype(mm),
                      ln_g, ln_b, w2.astype(mm), G=G, N=N)
    return out.reshape(B, N)


# speedup vs baseline: 1.3546x; 1.3546x over previous
import functools

import jax
import jax.numpy as jnp
from jax.experimental import pallas as pl
from jax.experimental.pallas import tpu as pltpu


def _gelu(x):
    c = 0.7978845608028654
    return 0.5 * x * (1.0 + jnp.tanh(c * (x + 0.044715 * x * x * x)))


_TRANS_RHS = (((1,), (1,)), ((), ()))


def _gcn_fused_kernel(adj_ref, x_ref, wdr_ref, w1_ref, g_ref, b_ref, w2_ref,
                      out_ref, *, G, N):
    H = wdr_ref.shape[0]
    GN = G * N
    mm = wdr_ref.dtype
    wdr = wdr_ref[...]
    x0 = x_ref[...].reshape(GN, H)

    adj2 = adj_ref[...].reshape(GN, N)
    sel = jax.lax.broadcasted_iota(jnp.int32, (N, GN), 1)
    lane = jax.lax.broadcasted_iota(jnp.int32, (N, GN), 0)
    expand = (sel % N == lane).astype(jnp.float32)
    tiled = jnp.dot(adj2, expand,
                    preferred_element_type=jnp.float32)
    row = jax.lax.broadcasted_iota(jnp.int32, (GN, GN), 0)
    col = jax.lax.broadcasted_iota(jnp.int32, (GN, GN), 1)
    bd = jnp.where((row // N) == (col // N), tiled, 0.0).astype(mm)
    _T_LHS = (((0,), (0,)), ((), ()))

    def layer(x_bf):
        y = jnp.dot(x_bf, wdr,
                    preferred_element_type=jnp.float32)
        d = _gelu(y[:, :H].astype(mm))
        r = y[:, H:]
        diff = jax.lax.dot_general(bd, d, _T_LHS,
                                   preferred_element_type=jnp.float32)
        return _gelu((r + diff).astype(mm))

    x = layer(layer(x0.astype(mm)))

    h = _gelu(jnp.dot(x, w1_ref[...],
                      preferred_element_type=jnp.float32).astype(mm))
    h = h.astype(jnp.float32)
    mu = jnp.mean(h, axis=-1, keepdims=True)
    var = jnp.mean((h - mu) ** 2, axis=-1, keepdims=True)
    h = ((h - mu) * jax.lax.rsqrt(var + 1e-5)
         * g_ref[...].astype(jnp.float32) + b_ref[...].astype(jnp.float32))
    out_ref[0] = jax.lax.dot_general(w2_ref[...], h.astype(mm), _TRANS_RHS,
                                     preferred_element_type=jnp.float32)


def _choose_group(B, N, target_rows=128):
    per = max(1, target_rows // N)
    if B >= 2:
        per = min(per, B // 2)
    per = max(1, per)
    while B % per:
        per -= 1
    return per


def _gcn_pallas(adj_b, sem_b, wdr_bf, w1_bf, ln_g, ln_b, w2_bf, *, G, N):
    B = adj_b.shape[0]
    H = sem_b.shape[-1]
    Bg, GN = B // G, G * N
    const2 = lambda b: (0, 0)
    once = pl.Buffered(buffer_count=1)
    return pl.pallas_call(
        functools.partial(_gcn_fused_kernel, G=G, N=N),
        out_shape=jax.ShapeDtypeStruct((Bg, 1, GN), jnp.float32),
        grid_spec=pltpu.PrefetchScalarGridSpec(
            num_scalar_prefetch=0,
            grid=(Bg,),
            in_specs=[
                pl.BlockSpec((G, N, N), lambda b: (b, 0, 0)),
                pl.BlockSpec((G, N, H), lambda b: (b, 0, 0)),
                pl.BlockSpec((H, 2 * H), const2, pipeline_mode=once),
                pl.BlockSpec((H, H), const2, pipeline_mode=once),
                pl.BlockSpec((1, H), const2, pipeline_mode=once),
                pl.BlockSpec((1, H), const2, pipeline_mode=once),
                pl.BlockSpec((1, H), const2, pipeline_mode=once),
            ],
            out_specs=pl.BlockSpec((1, 1, GN), lambda b: (b, 0, 0)),
        ),
        compiler_params=pltpu.CompilerParams(
            dimension_semantics=("parallel",),
            vmem_limit_bytes=64 * 1024 * 1024,
        ),
    )(adj_b, sem_b, wdr_bf, w1_bf, ln_g, ln_b, w2_bf)


def kernel(adj_b, sem_b, wdr_t, w1_t, ln_g, ln_b, w2):
    B, N, _ = adj_b.shape
    H = sem_b.shape[-1]
    G = _choose_group(B, N, target_rows=256)
    mm = jnp.bfloat16
    out = _gcn_pallas(adj_b, sem_b, wdr_t.astype(mm), w1_t.astype(mm),
                      ln_g, ln_b, w2.astype(mm), G=G, N=N)
    return out.reshape(B, N)


# speedup vs baseline: 1.4268x; 1.0533x over previous
import functools

import numpy as np

import jax
import jax.numpy as jnp
from jax.experimental import pallas as pl
from jax.experimental.pallas import tpu as pltpu


def _gelu(x):
    c = 0.7978845608028654
    return 0.5 * x * (1.0 + jnp.tanh(c * (x + 0.044715 * x * x * x)))


_TRANS_RHS = (((1,), (1,)), ((), ()))


def _gcn_fused_kernel(adj_ref, x_ref, mask_ref, wdr_ref, w1_ref, g_ref, b_ref,
                      w2_ref, out_ref, *, G, N):
    H = wdr_ref.shape[0]
    GN = G * N
    mm = wdr_ref.dtype
    wdr = wdr_ref[...]
    x0 = x_ref[...].reshape(GN, H)

    adj2 = adj_ref[...].reshape(GN, N)
    sel = jax.lax.broadcasted_iota(jnp.int32, (N, GN), 1)
    lane = jax.lax.broadcasted_iota(jnp.int32, (N, GN), 0)
    expand = (sel % N == lane).astype(jnp.float32)
    tiled = jnp.dot(adj2, expand,
                    preferred_element_type=jnp.float32)
    bd = tiled.astype(mm) * mask_ref[...]
    _T_LHS = (((0,), (0,)), ((), ()))

    def layer(x_bf):
        y = jnp.dot(x_bf, wdr,
                    preferred_element_type=jnp.float32)
        d = _gelu(y[:, :H].astype(mm))
        r = y[:, H:]
        diff = jax.lax.dot_general(bd, d, _T_LHS,
                                   preferred_element_type=jnp.float32)
        return _gelu((r + diff).astype(mm))

    x = layer(layer(x0.astype(mm)))

    h = _gelu(jnp.dot(x, w1_ref[...],
                      preferred_element_type=jnp.float32).astype(mm))
    h = h.astype(jnp.float32)
    mu = jnp.mean(h, axis=-1, keepdims=True)
    var = jnp.mean((h - mu) ** 2, axis=-1, keepdims=True)
    h = ((h - mu) * jax.lax.rsqrt(var + 1e-5)
         * g_ref[...].astype(jnp.float32) + b_ref[...].astype(jnp.float32))
    out_ref[0] = jax.lax.dot_general(w2_ref[...], h.astype(mm), _TRANS_RHS,
                                     preferred_element_type=jnp.float32)


def _choose_group(B, N, target_rows=128):
    per = max(1, target_rows // N)
    if B >= 2:
        per = min(per, B // 2)
    per = max(1, per)
    while B % per:
        per -= 1
    return per


def _gcn_pallas(adj_b, sem_b, mask_bf, wdr_bf, w1_bf, ln_g, ln_b, w2_bf,
                *, G, N):
    B = adj_b.shape[0]
    H = sem_b.shape[-1]
    Bg, GN = B // G, G * N
    const2 = lambda b: (0, 0)
    once = pl.Buffered(buffer_count=1)
    return pl.pallas_call(
        functools.partial(_gcn_fused_kernel, G=G, N=N),
        out_shape=jax.ShapeDtypeStruct((Bg, 1, GN), jnp.float32),
        grid_spec=pltpu.PrefetchScalarGridSpec(
            num_scalar_prefetch=0,
            grid=(Bg,),
            in_specs=[
                pl.BlockSpec((G, N, N), lambda b: (b, 0, 0)),
                pl.BlockSpec((G, N, H), lambda b: (b, 0, 0)),
                pl.BlockSpec((GN, GN), const2, pipeline_mode=once),
                pl.BlockSpec((H, 2 * H), const2, pipeline_mode=once),
                pl.BlockSpec((H, H), const2, pipeline_mode=once),
                pl.BlockSpec((1, H), const2, pipeline_mode=once),
                pl.BlockSpec((1, H), const2, pipeline_mode=once),
                pl.BlockSpec((1, H), const2, pipeline_mode=once),
            ],
            out_specs=pl.BlockSpec((1, 1, GN), lambda b: (b, 0, 0)),
        ),
        compiler_params=pltpu.CompilerParams(
            dimension_semantics=("parallel",),
            vmem_limit_bytes=64 * 1024 * 1024,
        ),
    )(adj_b, sem_b, mask_bf, wdr_bf, w1_bf, ln_g, ln_b, w2_bf)


def kernel(adj_b, sem_b, wdr_t, w1_t, ln_g, ln_b, w2):
    B, N, _ = adj_b.shape
    H = sem_b.shape[-1]
    G = _choose_group(B, N, target_rows=512)
    GN = G * N
    mm = jnp.bfloat16
    blk = np.arange(GN) // N
    mask_bf = jnp.asarray((blk[:, None] == blk[None, :]).astype(np.float32),
                          dtype=mm)
    out = _gcn_pallas(adj_b, sem_b, mask_bf, wdr_t.astype(mm),
                      w1_t.astype(mm), ln_g, ln_b, w2.astype(mm), G=G, N=N)
    return out.reshape(B, N)


# speedup vs baseline: 1.5559x; 1.0905x over previous
import functools

import numpy as np

import jax
import jax.numpy as jnp
from jax.experimental import pallas as pl
from jax.experimental.pallas import tpu as pltpu


def _gelu(x):
    c = 0.7978845608028654
    return 0.5 * x * (1.0 + jnp.tanh(c * (x + 0.044715 * x * x * x)))


_TRANS_RHS = (((1,), (1,)), ((), ()))


def _gcn_fused_kernel(adj_ref, x_ref, mask_ref, wdr_ref, w1_ref, g_ref, b_ref,
                      w2_ref, out_ref, *, G, N):
    H = wdr_ref.shape[0]
    GN = G * N
    mm = jnp.bfloat16
    wdr = wdr_ref[...].astype(mm)
    x0 = x_ref[...].reshape(GN, H)

    adj2 = adj_ref[...]
    sel = jax.lax.broadcasted_iota(jnp.int32, (N, GN), 1)
    lane = jax.lax.broadcasted_iota(jnp.int32, (N, GN), 0)
    expand = (sel % N == lane).astype(jnp.float32)
    tiled = jnp.dot(adj2, expand,
                    preferred_element_type=jnp.float32)
    bd = tiled.astype(mm) * mask_ref[...]
    _T_LHS = (((0,), (0,)), ((), ()))

    def layer(x_bf):
        y = jnp.dot(x_bf, wdr,
                    preferred_element_type=jnp.float32)
        d = _gelu(y[:, :H].astype(mm))
        r = y[:, H:]
        diff = jax.lax.dot_general(bd, d, _T_LHS,
                                   preferred_element_type=jnp.float32)
        return _gelu((r + diff).astype(mm))

    x = layer(layer(x0.astype(mm)))

    h = _gelu(jnp.dot(x, w1_ref[...].astype(mm),
                      preferred_element_type=jnp.float32).astype(mm))
    h = h.astype(jnp.float32)
    mu = jnp.mean(h, axis=-1, keepdims=True)
    var = jnp.mean(h * h, axis=-1, keepdims=True) - mu * mu
    h = ((h - mu) * jax.lax.rsqrt(var + 1e-5)
         * g_ref[...].astype(jnp.float32) + b_ref[...].astype(jnp.float32))
    out_ref[0] = jax.lax.dot_general(w2_ref[...].astype(mm), h.astype(mm),
                                     _TRANS_RHS,
                                     preferred_element_type=jnp.float32)


def _choose_group(B, N, target_rows=128):
    per = max(1, target_rows // N)
    if B >= 2:
        per = min(per, B // 2)
    per = max(1, per)
    while B % per:
        per -= 1
    return per


def _gcn_pallas(adj_b, sem_b, mask_bf, wdr_bf, w1_bf, ln_g, ln_b, w2_bf,
                *, G, N):
    B = adj_b.shape[0]
    H = sem_b.shape[-1]
    Bg, GN = B // G, G * N
    const2 = lambda b: (0, 0)
    once = pl.Buffered(buffer_count=1)
    return pl.pallas_call(
        functools.partial(_gcn_fused_kernel, G=G, N=N),
        out_shape=jax.ShapeDtypeStruct((Bg, 1, GN), jnp.float32),
        grid_spec=pltpu.PrefetchScalarGridSpec(
            num_scalar_prefetch=0,
            grid=(Bg,),
            in_specs=[
                pl.BlockSpec((GN, N), lambda b: (b, 0)),
                pl.BlockSpec((G, N, H), lambda b: (b, 0, 0)),
                pl.BlockSpec((GN, GN), const2, pipeline_mode=once),
                pl.BlockSpec((H, 2 * H), const2, pipeline_mode=once),
                pl.BlockSpec((H, H), const2, pipeline_mode=once),
                pl.BlockSpec((1, H), const2, pipeline_mode=once),
                pl.BlockSpec((1, H), const2, pipeline_mode=once),
                pl.BlockSpec((1, H), const2, pipeline_mode=once),
            ],
            out_specs=pl.BlockSpec((1, 1, GN), lambda b: (b, 0, 0)),
        ),
        compiler_params=pltpu.CompilerParams(
            dimension_semantics=("parallel",),
            vmem_limit_bytes=64 * 1024 * 1024,
        ),
    )(adj_b.reshape(B * N, N), sem_b, mask_bf, wdr_bf, w1_bf, ln_g, ln_b,
      w2_bf)


def kernel(adj_b, sem_b, wdr_t, w1_t, ln_g, ln_b, w2):
    B, N, _ = adj_b.shape
    H = sem_b.shape[-1]
    G = _choose_group(B, N, target_rows=512)
    GN = G * N
    mm = jnp.bfloat16
    blk = np.arange(GN) // N
    mask_bf = jnp.asarray((blk[:, None] == blk[None, :]).astype(np.float32),
                          dtype=mm)
    out = _gcn_pallas(adj_b, sem_b, mask_bf, wdr_t, w1_t, ln_g, ln_b, w2,
                      G=G, N=N)
    return out.reshape(B, N)


# speedup vs baseline: 1.5596x; 1.0023x over previous
import functools

import numpy as np

import jax
import jax.numpy as jnp
from jax.experimental import pallas as pl
from jax.experimental.pallas import tpu as pltpu


def _gelu(x):
    c = 0.7978845608028654
    return 0.5 * x * (1.0 + jnp.tanh(c * (x + 0.044715 * x * x * x)))


_TRANS_RHS = (((1,), (1,)), ((), ()))


def _gcn_fused_kernel(adj_ref, x_ref, mask_ref, wdr_ref, w1_ref, g_ref, b_ref,
                      w2_ref, out_ref, *, G, N):
    H = wdr_ref.shape[0]
    GN = G * N
    mm = jnp.bfloat16
    wdr = wdr_ref[...].astype(mm)
    w1 = w1_ref[...].astype(mm)
    w2 = w2_ref[...].astype(mm)
    gam = g_ref[...].astype(jnp.float32)
    bet = b_ref[...].astype(jnp.float32)
    mask = mask_ref[...]
    x0 = x_ref[...].reshape(GN, H)
    adj2 = adj_ref[...]
    sel = jax.lax.broadcasted_iota(jnp.int32, (N, GN), 1)
    lane = jax.lax.broadcasted_iota(jnp.int32, (N, GN), 0)
    expand = (sel % N == lane).astype(jnp.float32)
    _T_LHS = (((0,), (0,)), ((), ()))

    tiled = jnp.dot(adj2, expand,
                    preferred_element_type=jnp.float32)
    bd = tiled.astype(mm) * mask

    def layer(x_bf):
        y = jnp.dot(x_bf, wdr,
                    preferred_element_type=jnp.float32)
        d = _gelu(y[:, :H].astype(mm))
        r = y[:, H:]
        diff = jax.lax.dot_general(bd, d, _T_LHS,
                                   preferred_element_type=jnp.float32)
        return _gelu((r + diff).astype(mm))

    x = layer(layer(x0.astype(mm)))

    h = _gelu(jnp.dot(x, w1,
                      preferred_element_type=jnp.float32).astype(mm))
    h = h.astype(jnp.float32)
    mu = jnp.mean(h, axis=-1, keepdims=True)
    var = jnp.mean(h * h, axis=-1, keepdims=True) - mu * mu
    h = (h - mu) * jax.lax.rsqrt(var + 1e-5) * gam + bet
    out_ref[0] = jax.lax.dot_general(w2, h.astype(mm), _TRANS_RHS,
                                     preferred_element_type=jnp.float32)


def _choose_group(B, N, target_rows=128):
    per = max(1, target_rows // N)
    if B >= 2:
        per = min(per, B // 2)
    per = max(1, per)
    while B % per:
        per -= 1
    return per


def _gcn_pallas(adj_b, sem_b, mask_bf, wdr_bf, w1_bf, ln_g, ln_b, w2_bf,
                *, G, N):
    B = adj_b.shape[0]
    H = sem_b.shape[-1]
    Bg, GN = B // G, G * N
    const2 = lambda b: (0, 0)
    once = pl.Buffered(buffer_count=1)
    return pl.pallas_call(
        functools.partial(_gcn_fused_kernel, G=G, N=N),
        out_shape=jax.ShapeDtypeStruct((Bg, 1, GN), jnp.float32),
        grid_spec=pltpu.PrefetchScalarGridSpec(
            num_scalar_prefetch=0,
            grid=(Bg,),
            in_specs=[
                pl.BlockSpec((GN, N), lambda b: (b, 0)),
                pl.BlockSpec((G, N, H), lambda b: (b, 0, 0)),
                pl.BlockSpec((GN, GN), const2, pipeline_mode=once),
                pl.BlockSpec((H, 2 * H), const2, pipeline_mode=once),
                pl.BlockSpec((H, H), const2, pipeline_mode=once),
                pl.BlockSpec((1, H), const2, pipeline_mode=once),
                pl.BlockSpec((1, H), const2, pipeline_mode=once),
                pl.BlockSpec((1, H), const2, pipeline_mode=once),
            ],
            out_specs=pl.BlockSpec((1, 1, GN), lambda b: (b, 0, 0)),
        ),
        compiler_params=pltpu.CompilerParams(
            dimension_semantics=("parallel",),
            vmem_limit_bytes=64 * 1024 * 1024,
        ),
    )(adj_b.reshape(B * N, N), sem_b, mask_bf, wdr_bf, w1_bf, ln_g, ln_b,
      w2_bf)


def kernel(adj_b, sem_b, wdr_t, w1_t, ln_g, ln_b, w2):
    B, N, _ = adj_b.shape
    H = sem_b.shape[-1]
    G = _choose_group(B, N, target_rows=512)
    GN = G * N
    mm = jnp.bfloat16
    blk = np.arange(GN) // N
    mask_bf = jnp.asarray((blk[:, None] == blk[None, :]).astype(np.float32),
                          dtype=mm)
    out = _gcn_pallas(adj_b, sem_b, mask_bf, wdr_t, w1_t, ln_g, ln_b, w2,
                      G=G, N=N)
    return out.reshape(B, N)
